# baseline (speedup 1.0000x reference)
import functools

import jax
import jax.numpy as jnp
from jax import lax
from jax.experimental import pallas as pl
from jax.experimental.pallas import tpu as pltpu
from jax.experimental.pallas import tpu_sc as plsc

N = 10000
E = 160000
D_IN = 256
HIDDEN = 512
F = 64
KITER = 10
ALPHA = 0.1

NC = 2
NS = 16
NWALL = NC * NS
CE = 128
NCH2 = 40
EPW2 = NCH2 * CE
NCH1 = 80
EPW1 = NCH1 * CE
NP = 10112
RW = NP // NS
HALF = RW // 2
QR = RW // 4
ZROW = N
TROW = N + 1

_MESH1 = plsc.VectorSubcoreMesh(core_axis_name="c", subcore_axis_name="s",
                                num_cores=1)
_MESH2 = plsc.VectorSubcoreMesh(core_axis_name="c", subcore_axis_name="s",
                                num_cores=2)


def _enc_body(x_ref, w1_ref, b1_ref, w2_ref, b2_ref, wo_ref, y_ref):
    h = jnp.dot(x_ref[...], w1_ref[...], preferred_element_type=jnp.float32)
    h = jnp.maximum(h + b1_ref[...], 0.0)
    h = jnp.dot(h, w2_ref[...], preferred_element_type=jnp.float32)
    h = jnp.maximum(h + b2_ref[...], 0.0)
    y_ref[...] = jnp.dot(h, wo_ref[...], preferred_element_type=jnp.float32)


def _encoder(x, W1, b1, W2, b2, Wout):
    RB = 1000
    return pl.pallas_call(
        _enc_body,
        grid=(N // RB,),
        in_specs=[
            pl.BlockSpec((RB, D_IN), lambda i: (i, 0)),
            pl.BlockSpec((D_IN, HIDDEN), lambda i: (0, 0)),
            pl.BlockSpec((1, HIDDEN), lambda i: (0, 0)),
            pl.BlockSpec((HIDDEN, HIDDEN), lambda i: (0, 0)),
            pl.BlockSpec((1, HIDDEN), lambda i: (0, 0)),
            pl.BlockSpec((HIDDEN, F), lambda i: (0, 0)),
        ],
        out_specs=pl.BlockSpec((RB, F), lambda i: (i, 0)),
        out_shape=jax.ShapeDtypeStruct((N, F), jnp.float32),
    )(x, W1, b1.reshape(1, HIDDEN), W2, b2.reshape(1, HIDDEN), Wout)


def _prep_body(cs_ref, y_ref, bout_ref, u0_ref, a_ref, c_ref, af_ref, cf_ref):
    RB = cs_ref.shape[0]
    i = pl.program_id(0)
    rows = lax.broadcasted_iota(jnp.int32, (RB, 1), 0) + i * RB
    deg = cs_ref[:, 0:1] + 1.0
    dinv = jnp.where(rows < N, lax.rsqrt(deg), 0.0)
    yb = y_ref[...]
    u0 = dinv * yb
    u0_ref[...] = u0
    a_ref[...] = jnp.broadcast_to((1.0 - ALPHA) * dinv * dinv, (RB, F))
    c_ref[...] = ALPHA * u0
    af_ref[...] = jnp.broadcast_to((1.0 - ALPHA) * dinv, (RB, F))
    cf_ref[...] = ALPHA * yb + bout_ref[...]


def _prep(colsum, y_pad, bout):
    RB = 632
    sds = jax.ShapeDtypeStruct((NP, F), jnp.float32)
    blk = pl.BlockSpec((RB, F), lambda i: (i, 0))
    return pl.pallas_call(
        _prep_body,
        grid=(NP // RB,),
        in_specs=[blk, blk, pl.BlockSpec((1, F), lambda i: (0, 0))],
        out_specs=[blk] * 5,
        out_shape=[sds] * 5,
    )(colsum, y_pad, bout.reshape(1, F))


def _comb_body(p0_ref, p1_ref, u_ref, a_ref, c_ref, o_ref):
    o_ref[...] = (a_ref[...] * (p0_ref[...] + p1_ref[...] + u_ref[...])
                  + c_ref[...])


def _combine(p0, p1, u, aa, cc):
    RB = 1264
    blk = pl.BlockSpec((RB, F), lambda i: (i, 0))
    return pl.pallas_call(
        _comb_body,
        grid=(NP // RB,),
        in_specs=[blk] * 5,
        out_specs=blk,
        out_shape=jax.ShapeDtypeStruct((NP, F), jnp.float32),
    )(p0, p1, u, aa, cc)


def _fill(buf, nrows, value):
    vec = jnp.full((16,), value, jnp.float32)

    def body(r, _):
        for cc in range(F // 16):
            buf[r, pl.ds(cc * 16, 16)] = vec
        return 0

    lax.fori_loop(0, nrows, body, 0)


def _deg_body(dst_hbm, out_hbm, idx_v, ones_v, buf_v, s_sh):
    w = lax.axis_index("s")
    row0 = w * RW
    _fill(ones_v, CE, 1.0)
    _fill(buf_v, HALF, 0.0)
    for h in range(2):
        pltpu.sync_copy(buf_v, s_sh.at[pl.ds(row0 + h * HALF, HALF)])
    plsc.subcore_barrier()

    def chunk(j, _):
        pltpu.sync_copy(dst_hbm.at[w, j], idx_v)
        pltpu.sync_copy(ones_v, s_sh.at[idx_v], add=True)
        return 0

    lax.fori_loop(0, NCH1, chunk, 0)
    plsc.subcore_barrier()
    for h in range(2):
        r0 = row0 + h * HALF
        pltpu.sync_copy(s_sh.at[pl.ds(r0, HALF)], buf_v)
        pltpu.sync_copy(buf_v, out_hbm.at[pl.ds(r0, HALF)])


_deg_call = functools.partial(
    pl.kernel,
    out_type=jax.ShapeDtypeStruct((NP, F), jnp.float32),
    mesh=_MESH1,
    scratch_types=[
        pltpu.VMEM((CE,), jnp.int32),
        pltpu.VMEM((CE, F), jnp.float32),
        pltpu.VMEM((HALF, F), jnp.float32),
        pltpu.VMEM_SHARED((NP, F), jnp.float32),
    ],
    compiler_params=pltpu.CompilerParams(use_tc_tiling_on_sc=False),
)(_deg_body)


def _copy_idx4(dst, src2d, j):
    for r in range(4):
        for cc in range(CE // 16):
            dst[pl.ds(r * CE + cc * 16, 16)] = src2d[j + r, pl.ds(cc * 16, 16)]


def _scat_body(src_hbm, dst_hbm, u_hbm, p0_hbm, p1_hbm,
               idxs_v, idxd_v, idx_s0, idx_d0, idx_s1, idx_d1,
               rows0, rows1, buf_v, s_sh, sem0, sem1):
    c = lax.axis_index("c")
    s = lax.axis_index("s")
    wid = c * NS + s
    row0 = s * RW
    npair = NCH2 // 8

    pltpu.sync_copy(src_hbm.at[wid], idxs_v)
    pltpu.sync_copy(dst_hbm.at[wid], idxd_v)

    _fill(buf_v, QR, 0.0)
    for h in range(4):
        pltpu.sync_copy(buf_v, s_sh.at[pl.ds(row0 + h * QR, QR)])
    plsc.subcore_barrier()

    def pair(p, _):
        j0 = 8 * p
        _copy_idx4(idx_s0, idxs_v, j0)
        _copy_idx4(idx_d0, idxd_v, j0)
        _copy_idx4(idx_s1, idxs_v, j0 + 4)
        _copy_idx4(idx_d1, idxd_v, j0 + 4)
        d0 = pltpu.async_copy(u_hbm.at[idx_s0], rows0, sem0)
        d1 = pltpu.async_copy(u_hbm.at[idx_s1], rows1, sem1)
        d0.wait()
        pltpu.sync_copy(rows0, s_sh.at[idx_d0], add=True)
        d1.wait()
        pltpu.sync_copy(rows1, s_sh.at[idx_d1], add=True)
        return 0

    lax.fori_loop(0, npair, pair, 0)
    plsc.subcore_barrier()

    for h in range(4):
        r0 = row0 + h * QR
        pltpu.sync_copy(s_sh.at[pl.ds(r0, QR)], buf_v)

        @pl.when(c == 0)
        def _():
            pltpu.sync_copy(buf_v, p0_hbm.at[pl.ds(r0, QR)])

        @pl.when(c == 1)
        def _():
            pltpu.sync_copy(buf_v, p1_hbm.at[pl.ds(r0, QR)])


_scat_call = functools.partial(
    pl.kernel,
    out_type=[jax.ShapeDtypeStruct((NP, F), jnp.float32),
              jax.ShapeDtypeStruct((NP, F), jnp.float32)],
    mesh=_MESH2,
    scratch_types=(
        [pltpu.VMEM((NCH2, CE), jnp.int32)] * 2
        + [pltpu.VMEM((4 * CE,), jnp.int32)] * 4
        + [pltpu.VMEM((4 * CE, F), jnp.float32)] * 2
        + [pltpu.VMEM((QR, F), jnp.float32)]
        + [pltpu.VMEM_SHARED((NP, F), jnp.float32)]
        + [pltpu.SemaphoreType.DMA] * 2
    ),
    compiler_params=pltpu.CompilerParams(use_tc_tiling_on_sc=False),
)(_scat_body)


def kernel(x, edge_index, W1, b1, W2, b2, Wout, bout):
    src = edge_index[0].astype(jnp.int32)
    dst = edge_index[1].astype(jnp.int32)
    pad1 = NS * EPW1 - E
    dst_p1 = jnp.concatenate(
        [dst, jnp.full((pad1,), TROW, jnp.int32)]).reshape(NS, NCH1, CE)
    pad2 = NWALL * EPW2 - E
    src_p2 = jnp.concatenate(
        [src, jnp.full((pad2,), ZROW, jnp.int32)]).reshape(NWALL, NCH2, CE)
    dst_p2 = jnp.concatenate(
        [dst, jnp.full((pad2,), TROW, jnp.int32)]).reshape(NWALL, NCH2, CE)

    y = _encoder(x, W1, b1, W2, b2, Wout)
    colsum = _deg_call(dst_p1)
    y_pad = jnp.pad(y, ((0, NP - N), (0, 0)))
    u, a64, c64, af64, cf64 = _prep(colsum, y_pad, bout)
    for k in range(KITER):
        p0, p1 = _scat_call(src_p2, dst_p2, u)
        if k < KITER - 1:
            u = _combine(p0, p1, u, a64, c64)
        else:
            u = _combine(p0, p1, u, af64, cf64)
    return u[:N]


# speedup vs baseline: 1.0272x; 1.0272x over previous
import functools

import jax
import jax.numpy as jnp
from jax import lax
from jax.experimental import pallas as pl
from jax.experimental.pallas import tpu as pltpu
from jax.experimental.pallas import tpu_sc as plsc

N = 10000
E = 160000
D_IN = 256
HIDDEN = 512
F = 64
KITER = 10
ALPHA = 0.1

NC = 2
NS = 16
NWALL = NC * NS
CE = 128
NCH2 = 40
EPW2 = NCH2 * CE
NP = 10112
RW = NP // NS
HALF = RW // 2
QR = RW // 4
ZROW = N
TROW = N + 1

_MESH1 = plsc.VectorSubcoreMesh(core_axis_name="c", subcore_axis_name="s",
                                num_cores=1)
_MESH2 = plsc.VectorSubcoreMesh(core_axis_name="c", subcore_axis_name="s",
                                num_cores=2)


def _enc_body(x_ref, w1_ref, b1_ref, w2_ref, b2_ref, wo_ref, y_ref):
    h = jnp.dot(x_ref[...], w1_ref[...], preferred_element_type=jnp.float32)
    h = jnp.maximum(h + b1_ref[...], 0.0)
    h = jnp.dot(h, w2_ref[...], preferred_element_type=jnp.float32)
    h = jnp.maximum(h + b2_ref[...], 0.0)
    y_ref[...] = jnp.dot(h, wo_ref[...], preferred_element_type=jnp.float32)


def _encoder(x, W1, b1, W2, b2, Wout):
    RB = 1000
    return pl.pallas_call(
        _enc_body,
        grid=(N // RB,),
        in_specs=[
            pl.BlockSpec((RB, D_IN), lambda i: (i, 0)),
            pl.BlockSpec((D_IN, HIDDEN), lambda i: (0, 0)),
            pl.BlockSpec((1, HIDDEN), lambda i: (0, 0)),
            pl.BlockSpec((HIDDEN, HIDDEN), lambda i: (0, 0)),
            pl.BlockSpec((1, HIDDEN), lambda i: (0, 0)),
            pl.BlockSpec((HIDDEN, F), lambda i: (0, 0)),
        ],
        out_specs=pl.BlockSpec((RB, F), lambda i: (i, 0)),
        out_shape=jax.ShapeDtypeStruct((N, F), jnp.float32),
    )(x, W1, b1.reshape(1, HIDDEN), W2, b2.reshape(1, HIDDEN), Wout)


def _prep_body(cs0_ref, cs1_ref, y_ref, bout_ref, u0_ref, a_ref, c_ref,
               af_ref, cf_ref):
    RB = cs0_ref.shape[0]
    i = pl.program_id(0)
    rows = lax.broadcasted_iota(jnp.int32, (RB, 1), 0) + i * RB
    deg = cs0_ref[:, 0:1] + cs1_ref[:, 0:1] + 1.0
    dinv = jnp.where(rows < N, lax.rsqrt(deg), 0.0)
    yb = y_ref[...]
    u0 = dinv * yb
    u0_ref[...] = u0
    a_ref[...] = jnp.broadcast_to((1.0 - ALPHA) * dinv * dinv, (RB, F))
    c_ref[...] = ALPHA * u0
    af_ref[...] = jnp.broadcast_to((1.0 - ALPHA) * dinv, (RB, F))
    cf_ref[...] = ALPHA * yb + bout_ref[...]


def _prep(colsum0, colsum1, y_pad, bout):
    RB = 632
    sds = jax.ShapeDtypeStruct((NP, F), jnp.float32)
    blk = pl.BlockSpec((RB, F), lambda i: (i, 0))
    return pl.pallas_call(
        _prep_body,
        grid=(NP // RB,),
        in_specs=[blk, blk, blk, pl.BlockSpec((1, F), lambda i: (0, 0))],
        out_specs=[blk] * 5,
        out_shape=[sds] * 5,
    )(colsum0, colsum1, y_pad, bout.reshape(1, F))


def _comb_body(p0_ref, p1_ref, u_ref, a_ref, c_ref, o_ref):
    o_ref[...] = (a_ref[...] * (p0_ref[...] + p1_ref[...] + u_ref[...])
                  + c_ref[...])


def _combine(p0, p1, u, aa, cc):
    RB = 1264
    blk = pl.BlockSpec((RB, F), lambda i: (i, 0))
    return pl.pallas_call(
        _comb_body,
        grid=(NP // RB,),
        in_specs=[blk] * 5,
        out_specs=blk,
        out_shape=jax.ShapeDtypeStruct((NP, F), jnp.float32),
    )(p0, p1, u, aa, cc)


def _fill(buf, nrows, value):
    vec = jnp.full((16,), value, jnp.float32)

    def body(r, _):
        for cc in range(F // 16):
            buf[r, pl.ds(cc * 16, 16)] = vec
        return 0

    lax.fori_loop(0, nrows, body, 0)


def _deg_body(dst_hbm, p0_hbm, p1_hbm, idx_v, ones_v, buf_v, s_sh):
    c = lax.axis_index("c")
    sx = lax.axis_index("s")
    wid = c * NS + sx
    row0 = sx * RW
    _fill(ones_v, CE, 1.0)
    _fill(buf_v, HALF, 0.0)
    for h in range(2):
        pltpu.sync_copy(buf_v, s_sh.at[pl.ds(row0 + h * HALF, HALF)])
    plsc.subcore_barrier()

    def chunk(j, _):
        pltpu.sync_copy(dst_hbm.at[wid, j], idx_v)
        pltpu.sync_copy(ones_v, s_sh.at[idx_v], add=True)
        return 0

    lax.fori_loop(0, NCH2, chunk, 0)
    plsc.subcore_barrier()
    for h in range(2):
        r0 = row0 + h * HALF
        pltpu.sync_copy(s_sh.at[pl.ds(r0, HALF)], buf_v)

        @pl.when(c == 0)
        def _():
            pltpu.sync_copy(buf_v, p0_hbm.at[pl.ds(r0, HALF)])

        @pl.when(c == 1)
        def _():
            pltpu.sync_copy(buf_v, p1_hbm.at[pl.ds(r0, HALF)])


_deg_call = functools.partial(
    pl.kernel,
    out_type=[jax.ShapeDtypeStruct((NP, F), jnp.float32),
              jax.ShapeDtypeStruct((NP, F), jnp.float32)],
    mesh=_MESH2,
    scratch_types=[
        pltpu.VMEM((CE,), jnp.int32),
        pltpu.VMEM((CE, F), jnp.float32),
        pltpu.VMEM((HALF, F), jnp.float32),
        pltpu.VMEM_SHARED((NP, F), jnp.float32),
    ],
    compiler_params=pltpu.CompilerParams(use_tc_tiling_on_sc=False),
)(_deg_body)


def _copy_idx4(dst, src2d, j):
    for r in range(4):
        for cc in range(CE // 16):
            dst[pl.ds(r * CE + cc * 16, 16)] = src2d[j + r, pl.ds(cc * 16, 16)]


def _scat_body(src_hbm, dst_hbm, u_hbm, p0_hbm, p1_hbm,
               idxs_v, idxd_v, idx_s0, idx_d0, idx_s1, idx_d1,
               rows0, rows1, buf_v, s_sh, sem0, sem1):
    c = lax.axis_index("c")
    s = lax.axis_index("s")
    wid = c * NS + s
    row0 = s * RW
    npair = NCH2 // 8

    pltpu.sync_copy(src_hbm.at[wid], idxs_v)
    pltpu.sync_copy(dst_hbm.at[wid], idxd_v)

    _fill(buf_v, QR, 0.0)
    for h in range(4):
        pltpu.sync_copy(buf_v, s_sh.at[pl.ds(row0 + h * QR, QR)])
    plsc.subcore_barrier()

    def pair(p, _):
        j0 = 8 * p
        _copy_idx4(idx_s0, idxs_v, j0)
        _copy_idx4(idx_d0, idxd_v, j0)
        _copy_idx4(idx_s1, idxs_v, j0 + 4)
        _copy_idx4(idx_d1, idxd_v, j0 + 4)
        d0 = pltpu.async_copy(u_hbm.at[idx_s0], rows0, sem0)
        d1 = pltpu.async_copy(u_hbm.at[idx_s1], rows1, sem1)
        d0.wait()
        pltpu.sync_copy(rows0, s_sh.at[idx_d0], add=True)
        d1.wait()
        pltpu.sync_copy(rows1, s_sh.at[idx_d1], add=True)
        return 0

    lax.fori_loop(0, npair, pair, 0)
    plsc.subcore_barrier()

    for h in range(4):
        r0 = row0 + h * QR
        pltpu.sync_copy(s_sh.at[pl.ds(r0, QR)], buf_v)

        @pl.when(c == 0)
        def _():
            pltpu.sync_copy(buf_v, p0_hbm.at[pl.ds(r0, QR)])

        @pl.when(c == 1)
        def _():
            pltpu.sync_copy(buf_v, p1_hbm.at[pl.ds(r0, QR)])


_scat_call = functools.partial(
    pl.kernel,
    out_type=[jax.ShapeDtypeStruct((NP, F), jnp.float32),
              jax.ShapeDtypeStruct((NP, F), jnp.float32)],
    mesh=_MESH2,
    scratch_types=(
        [pltpu.VMEM((NCH2, CE), jnp.int32)] * 2
        + [pltpu.VMEM((4 * CE,), jnp.int32)] * 4
        + [pltpu.VMEM((4 * CE, F), jnp.float32)] * 2
        + [pltpu.VMEM((QR, F), jnp.float32)]
        + [pltpu.VMEM_SHARED((NP, F), jnp.float32)]
        + [pltpu.SemaphoreType.DMA] * 2
    ),
    compiler_params=pltpu.CompilerParams(use_tc_tiling_on_sc=False),
)(_scat_body)


def kernel(x, edge_index, W1, b1, W2, b2, Wout, bout):
    src = edge_index[0].astype(jnp.int32)
    dst = edge_index[1].astype(jnp.int32)
    pad2 = NWALL * EPW2 - E
    src_p2 = jnp.concatenate(
        [src, jnp.full((pad2,), ZROW, jnp.int32)]).reshape(NWALL, NCH2, CE)
    dst_p2 = jnp.concatenate(
        [dst, jnp.full((pad2,), TROW, jnp.int32)]).reshape(NWALL, NCH2, CE)

    y = _encoder(x, W1, b1, W2, b2, Wout)
    cs0, cs1 = _deg_call(dst_p2)
    y_pad = jnp.pad(y, ((0, NP - N), (0, 0)))
    u, a64, c64, af64, cf64 = _prep(cs0, cs1, y_pad, bout)
    for k in range(KITER):
        p0, p1 = _scat_call(src_p2, dst_p2, u)
        if k < KITER - 1:
            u = _combine(p0, p1, u, a64, c64)
        else:
            u = _combine(p0, p1, u, af64, cf64)
    return u[:N]
